# Initial kernel scaffold; baseline (speedup 1.0000x reference)
import jax
import jax.numpy as jnp
from jax.experimental import pallas as pl


def kernel(h, edge_f, edge_dt, dst_idx, Wq, bq, Wk, bk, Wv, bv, Wo, bo, ln_g, ln_b, time_w, time_b):
    raise NotImplementedError("write your pallas kernel here")


# speedup vs baseline: 2.9372x; 2.9372x over previous
import functools

import jax
import jax.numpy as jnp
from jax import lax
from jax.experimental import pallas as pl
from jax.experimental.pallas import tpu as pltpu
from jax.experimental.pallas import tpu_sc as plsc


def _qfull_body(h_ref, tbp_ref, wq1_ref, wq2_ref, bq_ref, out_ref):
    ztf = jnp.cos(tbp_ref[...])
    qc = jnp.dot(ztf, wq2_ref[...], preferred_element_type=jnp.float32)
    out_ref[...] = (
        jnp.dot(h_ref[...], wq1_ref[...], preferred_element_type=jnp.float32)
        + qc + bq_ref[...]
    )


def _edge_body(hs_ref, ef_ref, dt_ref, qg_ref, twp_ref, tbp_ref,
               wk1_ref, wk2_ref, wk3_ref, bk_ref,
               wv1_ref, wv2_ref, wv3_ref, bv_ref,
               vw_ref, ex_ref, *, dh):
    tf = jnp.cos(dt_ref[...] * twp_ref[...] + tbp_ref[...])
    hs = hs_ref[...]
    ef = ef_ref[...]
    k = (jnp.dot(hs, wk1_ref[...], preferred_element_type=jnp.float32)
         + jnp.dot(ef, wk2_ref[...], preferred_element_type=jnp.float32)
         + jnp.dot(tf, wk3_ref[...], preferred_element_type=jnp.float32)
         + bk_ref[...])
    v = (jnp.dot(hs, wv1_ref[...], preferred_element_type=jnp.float32)
         + jnp.dot(ef, wv2_ref[...], preferred_element_type=jnp.float32)
         + jnp.dot(tf, wv3_ref[...], preferred_element_type=jnp.float32)
         + bv_ref[...])
    qk = qg_ref[...] * k
    s0 = jnp.sum(qk[:, :dh], axis=1, keepdims=True)
    s1 = jnp.sum(qk[:, dh:], axis=1, keepdims=True)
    s0 = jnp.where(s0 >= 0.0, s0, 0.2 * s0)
    s1 = jnp.where(s1 >= 0.0, s1, 0.2 * s1)
    e0 = jnp.exp(s0)
    e1 = jnp.exp(s1)
    lane = lax.broadcasted_iota(jnp.int32, v.shape, 1)
    vw_ref[...] = v * jnp.where(lane < dh, e0, e1)
    ex_ref[...] = jnp.where(lane == 0, e0, jnp.where(lane == 1, e1, 0.0))


def _final_body(pv_ref, pe_ref, hd_ref, wo1_ref, wo2_ref, bo_ref,
                g_ref, b_ref, out_ref, *, dh):
    accv = pv_ref[0] + pv_ref[1]
    acce = pe_ref[0] + pe_ref[1]
    d0 = acce[:, 0:1]
    d1 = acce[:, 1:2]
    lane = lax.broadcasted_iota(jnp.int32, accv.shape, 1)
    den = jnp.where(lane < dh, d0, d1)
    agg = accv / jnp.maximum(den, 1e-16)
    rst = (jnp.dot(agg, wo1_ref[...], preferred_element_type=jnp.float32)
           + jnp.dot(hd_ref[...], wo2_ref[...], preferred_element_type=jnp.float32)
           + bo_ref[...])
    rst = jnp.maximum(rst, 0.0)
    mu = jnp.mean(rst, axis=-1, keepdims=True)
    xc = rst - mu
    var = jnp.mean(xc * xc, axis=-1, keepdims=True)
    out_ref[...] = xc / jnp.sqrt(var + 1e-5) * g_ref[...] + b_ref[...]


def _sc_gather(qfull, dst2d, dsttab, *, e_rows):
    info = plsc.get_sparse_core_info()
    nc, ns = info.num_cores, info.num_subcores
    nw = nc * ns
    n_dst, d = qfull.shape
    mesh = plsc.VectorSubcoreMesh(core_axis_name="c", subcore_axis_name="s")

    @functools.partial(
        pl.kernel, mesh=mesh,
        compiler_params=pltpu.CompilerParams(needs_layout_passes=False),
        out_type=jax.ShapeDtypeStruct((e_rows * 128, d), jnp.float32),
        scratch_types=[
            pltpu.VMEM((n_dst,), jnp.int32),
            pltpu.VMEM((128,), jnp.int32),
            pltpu.VMEM((128,), jnp.int32),
            pltpu.VMEM((128, d), jnp.float32),
            pltpu.SemaphoreType.DMA,
        ],
    )
    def k(qfull_hbm, dst2d_hbm, dsttab_hbm, qg_hbm, tab_v, di_v, idx2_v, qbuf, sem):
        w = lax.axis_index("s") * nc + lax.axis_index("c")
        pltpu.sync_copy(dsttab_hbm, tab_v)
        nt = (e_rows + nw - 1) // nw

        def body(t, _):
            row = w + nw * t

            @pl.when(row < e_rows)
            def _():
                pltpu.sync_copy(dst2d_hbm.at[row], di_v)
                for j in range(8):
                    ds = pl.ds(j * 16, 16)
                    idx2_v[ds] = plsc.load_gather(tab_v, [di_v[ds]])
                pltpu.async_copy(qfull_hbm.at[idx2_v], qbuf, sem).wait()
                pltpu.sync_copy(qbuf, qg_hbm.at[pl.ds(row * 128, 128)])

            return 0

        lax.fori_loop(0, nt, body, 0)

    return k(qfull, dst2d, dsttab)


def _sc_scatter(vw, dst2d, zv, *, e_rows, n_pad):
    info = plsc.get_sparse_core_info()
    nc, ns = info.num_cores, info.num_subcores
    d = vw.shape[1]
    rows_per_s = n_pad // ns
    mesh = plsc.VectorSubcoreMesh(core_axis_name="c", subcore_axis_name="s")

    @functools.partial(
        pl.kernel, mesh=mesh,
        compiler_params=pltpu.CompilerParams(needs_layout_passes=False),
        out_type=jax.ShapeDtypeStruct((nc, n_pad, d), jnp.float32),
        scratch_types=[
            pltpu.VMEM((128,), jnp.int32),
            pltpu.VMEM((128, d), jnp.float32),
            pltpu.VMEM_SHARED((n_pad, d), jnp.float32),
            pltpu.SemaphoreType.DMA,
        ],
    )
    def k(vw_hbm, dst2d_hbm, zv_hbm, pv_hbm, idx_v, vbuf, accv, sem):
        c = lax.axis_index("c")
        s = lax.axis_index("s")
        rs = pl.ds(s * rows_per_s, rows_per_s)
        pltpu.sync_copy(zv_hbm.at[rs], accv.at[rs])
        plsc.subcore_barrier()

        half = e_rows // nc
        nt = (half + ns - 1) // ns

        def body(t, _):
            off = s + ns * t

            @pl.when(off < half)
            def _():
                row = c * half + off
                pltpu.sync_copy(dst2d_hbm.at[row], idx_v)
                pltpu.sync_copy(vw_hbm.at[pl.ds(row * 128, 128)], vbuf)
                pltpu.sync_copy(vbuf, accv.at[idx_v], add=True)

            return 0

        lax.fori_loop(0, nt, body, 0)
        plsc.subcore_barrier()
        pltpu.sync_copy(accv.at[rs], pv_hbm.at[c, rs])

    return k(vw, dst2d, zv)


def kernel(h, edge_f, edge_dt, dst_idx, Wq, bq, Wk, bk, Wv, bv, Wo, bo,
           ln_g, ln_b, time_w, time_b):
    e = edge_f.shape[0]
    n_dst = h.shape[0] - e
    d_node = h.shape[1]
    d_edge = edge_f.shape[1]
    d_time = time_w.shape[0]
    d_out = Wq.shape[1]
    dh = d_out // 2
    tp = 128

    f32 = jnp.float32
    twp = jnp.zeros((1, tp), f32).at[0, :d_time].set(time_w)
    tbp = jnp.zeros((1, tp), f32).at[0, :d_time].set(time_b)
    wq1 = Wq[:d_node]
    wq2 = jnp.zeros((tp, d_out), f32).at[:d_time].set(Wq[d_node:])
    wk1, wk2 = Wk[:d_node], Wk[d_node:d_node + d_edge]
    wk3 = jnp.zeros((tp, d_out), f32).at[:d_time].set(Wk[d_node + d_edge:])
    wv1, wv2 = Wv[:d_node], Wv[d_node:d_node + d_edge]
    wv3 = jnp.zeros((tp, d_out), f32).at[:d_time].set(Wv[d_node + d_edge:])
    wo1, wo2 = Wo[:d_out], Wo[d_out:]
    bq2 = bq.reshape(1, d_out)
    bk2 = bk.reshape(1, d_out)
    bv2 = bv.reshape(1, d_out)
    bo2 = bo.reshape(1, d_out)
    g2 = ln_g.reshape(1, d_out)
    b2 = ln_b.reshape(1, d_out)

    h_dst = h[:n_dst]
    h_src = h[n_dst:]
    dtc = edge_dt.reshape(e, 1)
    dst2d = dst_idx.reshape(e // 128, 128)
    dsttab = dst_idx[:n_dst]

    bq_blk = 2000
    full = lambda shape: pl.BlockSpec(shape, lambda i: (0,) * len(shape))
    qfull = pl.pallas_call(
        _qfull_body,
        grid=(n_dst // bq_blk,),
        in_specs=[
            pl.BlockSpec((bq_blk, d_node), lambda i: (i, 0)),
            full((1, tp)), full((d_node, d_out)), full((tp, d_out)),
            full((1, d_out)),
        ],
        out_specs=pl.BlockSpec((bq_blk, d_out), lambda i: (i, 0)),
        out_shape=jax.ShapeDtypeStruct((n_dst, d_out), f32),
    )(h_dst, tbp, wq1, wq2, bq2)

    qg = _sc_gather(qfull, dst2d, dsttab, e_rows=e // 128)

    be = 2560
    vw, expad = pl.pallas_call(
        functools.partial(_edge_body, dh=dh),
        grid=(e // be,),
        in_specs=[
            pl.BlockSpec((be, d_node), lambda i: (i, 0)),
            pl.BlockSpec((be, d_edge), lambda i: (i, 0)),
            pl.BlockSpec((be, 1), lambda i: (i, 0)),
            pl.BlockSpec((be, d_out), lambda i: (i, 0)),
            full((1, tp)), full((1, tp)),
            full((d_node, d_out)), full((d_edge, d_out)), full((tp, d_out)),
            full((1, d_out)),
            full((d_node, d_out)), full((d_edge, d_out)), full((tp, d_out)),
            full((1, d_out)),
        ],
        out_specs=[
            pl.BlockSpec((be, d_out), lambda i: (i, 0)),
            pl.BlockSpec((be, d_out), lambda i: (i, 0)),
        ],
        out_shape=[
            jax.ShapeDtypeStruct((e, d_out), f32),
            jax.ShapeDtypeStruct((e, d_out), f32),
        ],
    )(h_src, edge_f, dtc, qg, twp, tbp,
      wk1, wk2, wk3, bk2, wv1, wv2, wv3, bv2)

    n_pad = ((n_dst + 127) // 128) * 128
    zv = jnp.zeros((n_pad, d_out), f32)
    pv = _sc_scatter(vw, dst2d, zv, e_rows=e // 128, n_pad=n_pad)
    pe = _sc_scatter(expad, dst2d, zv, e_rows=e // 128, n_pad=n_pad)

    bf = 2000
    out = pl.pallas_call(
        functools.partial(_final_body, dh=dh),
        grid=(n_dst // bf,),
        in_specs=[
            pl.BlockSpec((2, bf, d_out), lambda i: (0, i, 0)),
            pl.BlockSpec((2, bf, d_out), lambda i: (0, i, 0)),
            pl.BlockSpec((bf, d_node), lambda i: (i, 0)),
            full((d_out, d_out)), full((d_node, d_out)), full((1, d_out)),
            full((1, d_out)), full((1, d_out)),
        ],
        out_specs=pl.BlockSpec((bf, d_out), lambda i: (i, 0)),
        out_shape=jax.ShapeDtypeStruct((n_dst, d_out), f32),
    )(pv, pe, h_dst, wo1, wo2, bo2, g2, b2)
    return out


# speedup vs baseline: 3.0664x; 1.0440x over previous
import functools

import jax
import jax.numpy as jnp
from jax import lax
from jax.experimental import pallas as pl
from jax.experimental.pallas import tpu as pltpu
from jax.experimental.pallas import tpu_sc as plsc


def _qfull_body(h_ref, tbp_ref, wq1_ref, wq2_ref, bq_ref, out_ref):
    ztf = jnp.cos(tbp_ref[...])
    qc = jnp.dot(ztf, wq2_ref[...], preferred_element_type=jnp.float32)
    out_ref[...] = (
        jnp.dot(h_ref[...], wq1_ref[...], preferred_element_type=jnp.float32)
        + qc + bq_ref[...]
    )


def _edge_body(hs_ref, ef_ref, dt_ref, qg_ref, twp_ref, tbp_ref,
               wk1_ref, wk2_ref, wk3_ref, bk_ref,
               wv1_ref, wv2_ref, wv3_ref, bv_ref,
               vw_ref, ex_ref, *, dh):
    tf = jnp.cos(dt_ref[...] * twp_ref[...] + tbp_ref[...])
    hs = hs_ref[...]
    ef = ef_ref[...]
    k = (jnp.dot(hs, wk1_ref[...], preferred_element_type=jnp.float32)
         + jnp.dot(ef, wk2_ref[...], preferred_element_type=jnp.float32)
         + jnp.dot(tf, wk3_ref[...], preferred_element_type=jnp.float32)
         + bk_ref[...])
    v = (jnp.dot(hs, wv1_ref[...], preferred_element_type=jnp.float32)
         + jnp.dot(ef, wv2_ref[...], preferred_element_type=jnp.float32)
         + jnp.dot(tf, wv3_ref[...], preferred_element_type=jnp.float32)
         + bv_ref[...])
    qk = qg_ref[...] * k
    s0 = jnp.sum(qk[:, :dh], axis=1, keepdims=True)
    s1 = jnp.sum(qk[:, dh:], axis=1, keepdims=True)
    s0 = jnp.where(s0 >= 0.0, s0, 0.2 * s0)
    s1 = jnp.where(s1 >= 0.0, s1, 0.2 * s1)
    e0 = jnp.exp(s0)
    e1 = jnp.exp(s1)
    lane = lax.broadcasted_iota(jnp.int32, v.shape, 1)
    vw_ref[...] = v * jnp.where(lane < dh, e0, e1)
    ex_ref[...] = jnp.where(lane == 0, e0, jnp.where(lane == 1, e1, 0.0))


def _final_body(pv_ref, pe_ref, hd_ref, wo1_ref, wo2_ref, bo_ref,
                g_ref, b_ref, out_ref, *, dh):
    accv = pv_ref[0] + pv_ref[1]
    acce = pe_ref[0] + pe_ref[1]
    d0 = acce[:, 0:1]
    d1 = acce[:, 1:2]
    lane = lax.broadcasted_iota(jnp.int32, accv.shape, 1)
    den = jnp.where(lane < dh, d0, d1)
    agg = accv / jnp.maximum(den, 1e-16)
    rst = (jnp.dot(agg, wo1_ref[...], preferred_element_type=jnp.float32)
           + jnp.dot(hd_ref[...], wo2_ref[...], preferred_element_type=jnp.float32)
           + bo_ref[...])
    rst = jnp.maximum(rst, 0.0)
    mu = jnp.mean(rst, axis=-1, keepdims=True)
    xc = rst - mu
    var = jnp.mean(xc * xc, axis=-1, keepdims=True)
    out_ref[...] = xc / jnp.sqrt(var + 1e-5) * g_ref[...] + b_ref[...]


def _sc_gather(qfull, dst2d, dsttab, *, e_rows):
    info = plsc.get_sparse_core_info()
    nc, ns = info.num_cores, info.num_subcores
    nw = nc * ns
    n_dst, d = qfull.shape
    mesh = plsc.VectorSubcoreMesh(core_axis_name="c", subcore_axis_name="s")

    base = (e_rows // nw) & ~1
    nbig = (e_rows - nw * base) // 2

    @functools.partial(
        pl.kernel, mesh=mesh,
        compiler_params=pltpu.CompilerParams(needs_layout_passes=False),
        out_type=jax.ShapeDtypeStruct((e_rows * 128, d), jnp.float32),
        scratch_types=[
            pltpu.VMEM((n_dst,), jnp.int32),
            pltpu.VMEM((2, 128), jnp.int32),
            pltpu.VMEM((2, 128), jnp.int32),
            pltpu.VMEM((256, d), jnp.float32),
            pltpu.SemaphoreType.DMA,
            pltpu.SemaphoreType.DMA,
            pltpu.SemaphoreType.DMA,
            pltpu.SemaphoreType.DMA,
        ],
    )
    def k(qfull_hbm, dst2d_hbm, dsttab_hbm, qg_hbm, tab_v, di_v, idx2_v, qbuf,
          sem_i, sem_g0, sem_g1, sem_s):
        w = lax.axis_index("s") * nc + lax.axis_index("c")
        pltpu.sync_copy(dsttab_hbm, tab_v)
        lo = jnp.where(w < nbig, w * (base + 2), nbig * 2 + w * base)
        np2 = jnp.where(w < nbig, (base + 2) // 2, base // 2)
        half0 = qbuf.at[pl.ds(0, 128)]
        half1 = qbuf.at[pl.ds(128, 128)]

        def body(p, _):
            q0 = lo + 2 * p
            pltpu.async_copy(dst2d_hbm.at[pl.ds(q0, 2)], di_v, sem_i).wait()
            for r in range(2):
                for j in range(8):
                    ds = pl.ds(j * 16, 16)
                    idx2_v[r, ds] = plsc.load_gather(tab_v, [di_v[r, ds]])

            @pl.when(p > 0)
            def _():
                pltpu.make_async_copy(
                    qbuf, qg_hbm.at[pl.ds((q0 - 2) * 128, 256)], sem_s).wait()

            g0 = pltpu.async_copy(qfull_hbm.at[idx2_v.at[0]], half0, sem_g0)
            g1 = pltpu.async_copy(qfull_hbm.at[idx2_v.at[1]], half1, sem_g1)
            g0.wait()
            g1.wait()
            pltpu.make_async_copy(
                qbuf, qg_hbm.at[pl.ds(q0 * 128, 256)], sem_s).start()
            return q0

        last = lax.fori_loop(0, np2, body, 0)
        pltpu.make_async_copy(
            qbuf, qg_hbm.at[pl.ds(last * 128, 256)], sem_s).wait()

    return k(qfull, dst2d, dsttab)


def _sc_scatter(vw, dst2d, zv, *, e_rows, n_pad):
    info = plsc.get_sparse_core_info()
    nc, ns = info.num_cores, info.num_subcores
    d = vw.shape[1]
    rows_per_s = n_pad // ns
    mesh = plsc.VectorSubcoreMesh(core_axis_name="c", subcore_axis_name="s")

    half = e_rows // nc
    base = (half // ns) & ~1
    nbig = (half - ns * base) // 2

    @functools.partial(
        pl.kernel, mesh=mesh,
        compiler_params=pltpu.CompilerParams(needs_layout_passes=False),
        out_type=jax.ShapeDtypeStruct((nc, n_pad, d), jnp.float32),
        scratch_types=[
            pltpu.VMEM((2, 128), jnp.int32),
            pltpu.VMEM((256, d), jnp.float32),
            pltpu.VMEM_SHARED((n_pad, d), jnp.float32),
            pltpu.SemaphoreType.DMA,
            pltpu.SemaphoreType.DMA,
            pltpu.SemaphoreType.DMA,
            pltpu.SemaphoreType.DMA,
        ],
    )
    def k(vw_hbm, dst2d_hbm, zv_hbm, pv_hbm, idx_v, vbuf, accv,
          sem_i, sem_v, sem_s0, sem_s1):
        c = lax.axis_index("c")
        s = lax.axis_index("s")
        rs = pl.ds(s * rows_per_s, rows_per_s)
        pltpu.sync_copy(zv_hbm.at[rs], accv.at[rs])
        plsc.subcore_barrier()

        lo = c * half + jnp.where(s < nbig, s * (base + 2), nbig * 2 + s * base)
        np2 = jnp.where(s < nbig, (base + 2) // 2, base // 2)
        half0 = vbuf.at[pl.ds(0, 128)]
        half1 = vbuf.at[pl.ds(128, 128)]

        def body(p, _):
            q0 = lo + 2 * p

            @pl.when(p > 0)
            def _():
                pltpu.make_async_copy(half0, accv.at[idx_v.at[0]], sem_s0).wait()
                pltpu.make_async_copy(half1, accv.at[idx_v.at[1]], sem_s1).wait()

            i = pltpu.async_copy(dst2d_hbm.at[pl.ds(q0, 2)], idx_v, sem_i)
            v = pltpu.async_copy(vw_hbm.at[pl.ds(q0 * 128, 256)], vbuf, sem_v)
            i.wait()
            v.wait()
            pltpu.async_copy(half0, accv.at[idx_v.at[0]], sem_s0, add=True)
            pltpu.async_copy(half1, accv.at[idx_v.at[1]], sem_s1, add=True)
            return 0

        lax.fori_loop(0, np2, body, 0)
        pltpu.make_async_copy(half0, accv.at[idx_v.at[0]], sem_s0).wait()
        pltpu.make_async_copy(half1, accv.at[idx_v.at[1]], sem_s1).wait()
        plsc.subcore_barrier()
        pltpu.sync_copy(accv.at[rs], pv_hbm.at[c, rs])

    return k(vw, dst2d, zv)


def kernel(h, edge_f, edge_dt, dst_idx, Wq, bq, Wk, bk, Wv, bv, Wo, bo,
           ln_g, ln_b, time_w, time_b):
    e = edge_f.shape[0]
    n_dst = h.shape[0] - e
    d_node = h.shape[1]
    d_edge = edge_f.shape[1]
    d_time = time_w.shape[0]
    d_out = Wq.shape[1]
    dh = d_out // 2
    tp = 128

    f32 = jnp.float32
    twp = jnp.zeros((1, tp), f32).at[0, :d_time].set(time_w)
    tbp = jnp.zeros((1, tp), f32).at[0, :d_time].set(time_b)
    wq1 = Wq[:d_node]
    wq2 = jnp.zeros((tp, d_out), f32).at[:d_time].set(Wq[d_node:])
    wk1, wk2 = Wk[:d_node], Wk[d_node:d_node + d_edge]
    wk3 = jnp.zeros((tp, d_out), f32).at[:d_time].set(Wk[d_node + d_edge:])
    wv1, wv2 = Wv[:d_node], Wv[d_node:d_node + d_edge]
    wv3 = jnp.zeros((tp, d_out), f32).at[:d_time].set(Wv[d_node + d_edge:])
    wo1, wo2 = Wo[:d_out], Wo[d_out:]
    bq2 = bq.reshape(1, d_out)
    bk2 = bk.reshape(1, d_out)
    bv2 = bv.reshape(1, d_out)
    bo2 = bo.reshape(1, d_out)
    g2 = ln_g.reshape(1, d_out)
    b2 = ln_b.reshape(1, d_out)

    h_dst = h[:n_dst]
    h_src = h[n_dst:]
    dtc = edge_dt.reshape(e, 1)
    dst2d = dst_idx.reshape(e // 128, 128)
    dsttab = dst_idx[:n_dst]

    bq_blk = 2000
    full = lambda shape: pl.BlockSpec(shape, lambda i: (0,) * len(shape))
    qfull = pl.pallas_call(
        _qfull_body,
        grid=(n_dst // bq_blk,),
        in_specs=[
            pl.BlockSpec((bq_blk, d_node), lambda i: (i, 0)),
            full((1, tp)), full((d_node, d_out)), full((tp, d_out)),
            full((1, d_out)),
        ],
        out_specs=pl.BlockSpec((bq_blk, d_out), lambda i: (i, 0)),
        out_shape=jax.ShapeDtypeStruct((n_dst, d_out), f32),
    )(h_dst, tbp, wq1, wq2, bq2)

    qg = _sc_gather(qfull, dst2d, dsttab, e_rows=e // 128)

    be = 2560
    vw, expad = pl.pallas_call(
        functools.partial(_edge_body, dh=dh),
        grid=(e // be,),
        in_specs=[
            pl.BlockSpec((be, d_node), lambda i: (i, 0)),
            pl.BlockSpec((be, d_edge), lambda i: (i, 0)),
            pl.BlockSpec((be, 1), lambda i: (i, 0)),
            pl.BlockSpec((be, d_out), lambda i: (i, 0)),
            full((1, tp)), full((1, tp)),
            full((d_node, d_out)), full((d_edge, d_out)), full((tp, d_out)),
            full((1, d_out)),
            full((d_node, d_out)), full((d_edge, d_out)), full((tp, d_out)),
            full((1, d_out)),
        ],
        out_specs=[
            pl.BlockSpec((be, d_out), lambda i: (i, 0)),
            pl.BlockSpec((be, d_out), lambda i: (i, 0)),
        ],
        out_shape=[
            jax.ShapeDtypeStruct((e, d_out), f32),
            jax.ShapeDtypeStruct((e, d_out), f32),
        ],
    )(h_src, edge_f, dtc, qg, twp, tbp,
      wk1, wk2, wk3, bk2, wv1, wv2, wv3, bv2)

    n_pad = ((n_dst + 127) // 128) * 128
    zv = jnp.zeros((n_pad, d_out), f32)
    pv = _sc_scatter(vw, dst2d, zv, e_rows=e // 128, n_pad=n_pad)
    pe = _sc_scatter(expad, dst2d, zv, e_rows=e // 128, n_pad=n_pad)

    bf = 2000
    out = pl.pallas_call(
        functools.partial(_final_body, dh=dh),
        grid=(n_dst // bf,),
        in_specs=[
            pl.BlockSpec((2, bf, d_out), lambda i: (0, i, 0)),
            pl.BlockSpec((2, bf, d_out), lambda i: (0, i, 0)),
            pl.BlockSpec((bf, d_node), lambda i: (i, 0)),
            full((d_out, d_out)), full((d_node, d_out)), full((1, d_out)),
            full((1, d_out)), full((1, d_out)),
        ],
        out_specs=pl.BlockSpec((bf, d_out), lambda i: (i, 0)),
        out_shape=jax.ShapeDtypeStruct((n_dst, d_out), f32),
    )(pv, pe, h_dst, wo1, wo2, bo2, g2, b2)
    return out
